# Initial kernel scaffold; baseline (speedup 1.0000x reference)
import functools

import jax
import jax.numpy as jnp
from jax import lax
from jax.experimental import pallas as pl
from jax.experimental.pallas import tpu as pltpu
from jax.experimental.pallas import tpu_sc as plsc

N_NODES = 10000
N_EDGES = 320000
D = 128

NC = 2
NS = 16
NW = NC * NS

K = 128
N_CHUNKS = N_EDGES // K
CHUNKS_PER_W = -(-N_CHUNKS // NW)
RPS = N_NODES // NS


def _sc_mesh():
    return plsc.VectorSubcoreMesh(core_axis_name="c", subcore_axis_name="s")


def _sc_deg_body(col2d, ew2d, zeros_n, degp, col_v, ew_v, deg_sh):
    c = lax.axis_index("c")
    s = lax.axis_index("s")
    wid = c * NS + s

    @pl.when(s == 0)
    def _init():
        pltpu.sync_copy(zeros_n, deg_sh)

    plsc.subcore_barrier()

    def chunk(i, carry):
        r = wid + i * NW

        @pl.when(r < N_CHUNKS)
        def _():
            pltpu.sync_copy(col2d.at[r], col_v.at[0])
            pltpu.sync_copy(ew2d.at[r], ew_v)
            pltpu.sync_copy(ew_v, deg_sh.at[col_v.at[0]], add=True)

        return carry

    lax.fori_loop(0, CHUNKS_PER_W, chunk, 0)
    plsc.subcore_barrier()

    @pl.when(s == 0)
    def _writeout():
        pltpu.sync_copy(deg_sh, degp.at[c])


def _sc_degree(col2d, ew2d, zeros_n):
    return pl.kernel(
        _sc_deg_body,
        out_type=jax.ShapeDtypeStruct((NC, N_NODES), jnp.float32),
        mesh=_sc_mesh(),
        scratch_types=[
            pltpu.VMEM((1, K), jnp.int32),
            pltpu.VMEM((K,), jnp.float32),
            pltpu.VMEM_SHARED((N_NODES,), jnp.float32),
        ],
    )(col2d, ew2d, zeros_n)


def _sc_edge_body(row2d, col2d, ew2d, y_hbm, zeros_nd, accp,
                  row_v, col_v, ew_v, rows_v, sem, acc_sh):
    c = lax.axis_index("c")
    s = lax.axis_index("s")
    wid = c * NS + s

    pltpu.sync_copy(zeros_nd.at[pl.ds(s * RPS, RPS)],
                    acc_sh.at[pl.ds(s * RPS, RPS)])
    plsc.subcore_barrier()

    def chunk(i, carry):
        r = wid + i * NW

        @pl.when(r < N_CHUNKS)
        def _():
            pltpu.sync_copy(row2d.at[r], row_v.at[0])
            pltpu.sync_copy(col2d.at[r], col_v.at[0])
            pltpu.sync_copy(ew2d.at[r], ew_v)
            pltpu.async_copy(y_hbm.at[row_v.at[0]], rows_v, sem).wait()

            def scale(e, c2):
                w = plsc.load_gather(ew_v, [jnp.full((16,), e, jnp.int32)])
                for d in range(D // 16):
                    rows_v[e, pl.ds(d * 16, 16)] = (
                        rows_v[e, pl.ds(d * 16, 16)] * w)
                return c2

            lax.fori_loop(0, K, scale, 0)
            pltpu.sync_copy(rows_v, acc_sh.at[col_v.at[0]], add=True)

        return carry

    lax.fori_loop(0, CHUNKS_PER_W, chunk, 0)
    plsc.subcore_barrier()

    pltpu.sync_copy(acc_sh.at[pl.ds(s * RPS, RPS)],
                    accp.at[c, pl.ds(s * RPS, RPS)])


def _sc_edge(row2d, col2d, ew2d, y, zeros_nd):
    return pl.kernel(
        _sc_edge_body,
        out_type=jax.ShapeDtypeStruct((NC, N_NODES, D), jnp.float32),
        mesh=_sc_mesh(),
        scratch_types=[
            pltpu.VMEM((1, K), jnp.int32),
            pltpu.VMEM((1, K), jnp.int32),
            pltpu.VMEM((K,), jnp.float32),
            pltpu.VMEM((K, D), jnp.float32),
            pltpu.SemaphoreType.DMA,
            pltpu.VMEM_SHARED((N_NODES, D), jnp.float32),
        ],
    )(row2d, col2d, ew2d, y, zeros_nd)


def _tc1_body(x_ref, w_ref, degt_ref, y_ref, dinv_ref):
    deg = degt_ref[:, 0:1] + degt_ref[:, 1:2] + 1.0
    dinv = lax.rsqrt(deg)
    xw = jnp.dot(x_ref[...], w_ref[...], preferred_element_type=jnp.float32)
    y_ref[...] = xw * dinv
    dinv_ref[...] = dinv


def _tc1(x, w_conv, degt):
    R = 2000
    grid = N_NODES // R
    return pl.pallas_call(
        _tc1_body,
        grid=(grid,),
        in_specs=[
            pl.BlockSpec((R, D), lambda i: (i, 0)),
            pl.BlockSpec((D, D), lambda i: (0, 0)),
            pl.BlockSpec((R, NC), lambda i: (i, 0)),
        ],
        out_specs=[
            pl.BlockSpec((R, D), lambda i: (i, 0)),
            pl.BlockSpec((R, 1), lambda i: (i, 0)),
        ],
        out_shape=[
            jax.ShapeDtypeStruct((N_NODES, D), jnp.float32),
            jax.ShapeDtypeStruct((N_NODES, 1), jnp.float32),
        ],
    )(x, w_conv, degt)


def _tc2_body(accp_ref, y_ref, dinv_ref, bc_ref, wf_ref, bf_ref,
              wf2_ref, bf2_ref, out_ref):
    a = accp_ref[0] + accp_ref[1] + y_ref[...]
    h = jnp.maximum(a * dinv_ref[...] + bc_ref[...], 0.0)
    h = jnp.maximum(
        jnp.dot(h, wf_ref[...], preferred_element_type=jnp.float32)
        + bf_ref[...], 0.0)
    out_ref[...] = (
        jnp.dot(h, wf2_ref[...], preferred_element_type=jnp.float32)
        + bf2_ref[...])


def _tc2(accp, y, dinv, b_conv, w_fc, b_fc, w_fc2, b_fc2):
    R = 2000
    grid = N_NODES // R
    wspec = pl.BlockSpec((D, D), lambda i: (0, 0))
    bspec = pl.BlockSpec((1, D), lambda i: (0, 0))
    return pl.pallas_call(
        _tc2_body,
        grid=(grid,),
        in_specs=[
            pl.BlockSpec((NC, R, D), lambda i: (0, i, 0)),
            pl.BlockSpec((R, D), lambda i: (i, 0)),
            pl.BlockSpec((R, 1), lambda i: (i, 0)),
            bspec, wspec, bspec, wspec, bspec,
        ],
        out_specs=pl.BlockSpec((R, D), lambda i: (i, 0)),
        out_shape=jax.ShapeDtypeStruct((N_NODES, D), jnp.float32),
    )(accp, y, dinv, b_conv, w_fc, b_fc, w_fc2, b_fc2)


def kernel(g, x, edge_weight, W_conv, b_conv, W_fc, b_fc, W_fc2, b_fc2):
    row2d = g[0].astype(jnp.int32).reshape(N_CHUNKS, K)
    col2d = g[1].astype(jnp.int32).reshape(N_CHUNKS, K)
    ew2d = edge_weight.astype(jnp.float32).reshape(N_CHUNKS, K)

    zeros_n = jnp.zeros((N_NODES,), jnp.float32)
    zeros_nd = jnp.zeros((N_NODES, D), jnp.float32)

    degp = _sc_degree(col2d, ew2d, zeros_n)
    degt = degp.T
    y, dinv = _tc1(x, W_conv, degt)
    accp = _sc_edge(row2d, col2d, ew2d, y, zeros_nd)
    out = _tc2(accp, y, dinv,
               b_conv.reshape(1, D), W_fc, b_fc.reshape(1, D),
               W_fc2, b_fc2.reshape(1, D))
    return out


# speedup vs baseline: 14.7055x; 14.7055x over previous
import functools

import jax
import jax.numpy as jnp
from jax import lax
from jax.experimental import pallas as pl
from jax.experimental.pallas import tpu as pltpu
from jax.experimental.pallas import tpu_sc as plsc

N_NODES = 10000
N_EDGES = 320000
D = 128

NC = 2
NS = 16
NW = NC * NS

K = 128
N_CHUNKS = N_EDGES // K
CHUNKS_PER_W = -(-N_CHUNKS // NW)
RPS = N_NODES // NS


def _sc_mesh():
    return plsc.VectorSubcoreMesh(core_axis_name="c", subcore_axis_name="s")


_SC_PARAMS = pltpu.CompilerParams(use_tc_tiling_on_sc=False,
                                  needs_layout_passes=False)


def _sc_deg_body(col2d, ew2d, zeros_n, degp, col_v, ew_v, deg_sh):
    c = lax.axis_index("c")
    s = lax.axis_index("s")
    wid = c * NS + s

    @pl.when(s == 0)
    def _init():
        pltpu.sync_copy(zeros_n, deg_sh)

    plsc.subcore_barrier()

    def chunk(i, carry):
        r = wid + i * NW

        @pl.when(r < N_CHUNKS)
        def _():
            pltpu.sync_copy(col2d.at[r], col_v.at[0])
            pltpu.sync_copy(ew2d.at[r], ew_v)
            pltpu.sync_copy(ew_v, deg_sh.at[col_v.at[0]], add=True)

        return carry

    lax.fori_loop(0, CHUNKS_PER_W, chunk, 0)
    plsc.subcore_barrier()

    @pl.when(s == 0)
    def _writeout():
        pltpu.sync_copy(deg_sh, degp.at[c])


def _sc_degree(col2d, ew2d, zeros_n):
    return pl.kernel(
        _sc_deg_body,
        out_type=jax.ShapeDtypeStruct((NC, N_NODES), jnp.float32),
        mesh=_sc_mesh(),
        scratch_types=[
            pltpu.VMEM((1, K), jnp.int32),
            pltpu.VMEM((K,), jnp.float32),
            pltpu.VMEM_SHARED((N_NODES,), jnp.float32),
        ],
        compiler_params=_SC_PARAMS,
    )(col2d, ew2d, zeros_n)


def _sc_edge_body(row2d, col2d, ew2d, y_hbm, zeros_nd, accp,
                  row_v, col_v, ew_v, rows_v, sem, acc_sh):
    c = lax.axis_index("c")
    s = lax.axis_index("s")
    wid = c * NS + s

    pltpu.sync_copy(zeros_nd.at[pl.ds(s * RPS, RPS)],
                    acc_sh.at[pl.ds(s * RPS, RPS)])
    plsc.subcore_barrier()

    def chunk(i, carry):
        r = wid + i * NW

        @pl.when(r < N_CHUNKS)
        def _():
            pltpu.sync_copy(row2d.at[r], row_v.at[0])
            pltpu.sync_copy(col2d.at[r], col_v.at[0])
            pltpu.sync_copy(ew2d.at[r], ew_v)
            pltpu.async_copy(y_hbm.at[row_v.at[0]], rows_v, sem).wait()

            def scale(e, c2):
                w = plsc.load_gather(ew_v, [jnp.full((16,), e, jnp.int32)])
                for d in range(D // 16):
                    rows_v[e, pl.ds(d * 16, 16)] = (
                        rows_v[e, pl.ds(d * 16, 16)] * w)
                return c2

            lax.fori_loop(0, K, scale, 0)
            pltpu.sync_copy(rows_v, acc_sh.at[col_v.at[0]], add=True)

        return carry

    lax.fori_loop(0, CHUNKS_PER_W, chunk, 0)
    plsc.subcore_barrier()

    pltpu.sync_copy(acc_sh.at[pl.ds(s * RPS, RPS)],
                    accp.at[c, pl.ds(s * RPS, RPS)])


def _sc_edge(row2d, col2d, ew2d, y, zeros_nd):
    return pl.kernel(
        _sc_edge_body,
        out_type=jax.ShapeDtypeStruct((NC, N_NODES, D), jnp.float32),
        mesh=_sc_mesh(),
        scratch_types=[
            pltpu.VMEM((1, K), jnp.int32),
            pltpu.VMEM((1, K), jnp.int32),
            pltpu.VMEM((K,), jnp.float32),
            pltpu.VMEM((K, D), jnp.float32),
            pltpu.SemaphoreType.DMA,
            pltpu.VMEM_SHARED((N_NODES, D), jnp.float32),
        ],
        compiler_params=_SC_PARAMS,
    )(row2d, col2d, ew2d, y, zeros_nd)


def _tc1_body(x_ref, w_ref, degt_ref, y_ref, dinv_ref):
    deg = degt_ref[:, 0:1] + degt_ref[:, 1:2] + 1.0
    dinv = lax.rsqrt(deg)
    xw = jnp.dot(x_ref[...], w_ref[...], preferred_element_type=jnp.float32)
    y_ref[...] = xw * dinv
    dinv_ref[...] = dinv


def _tc1(x, w_conv, degt):
    R = 2000
    grid = N_NODES // R
    return pl.pallas_call(
        _tc1_body,
        grid=(grid,),
        in_specs=[
            pl.BlockSpec((R, D), lambda i: (i, 0)),
            pl.BlockSpec((D, D), lambda i: (0, 0)),
            pl.BlockSpec((R, NC), lambda i: (i, 0)),
        ],
        out_specs=[
            pl.BlockSpec((R, D), lambda i: (i, 0)),
            pl.BlockSpec((R, 1), lambda i: (i, 0)),
        ],
        out_shape=[
            jax.ShapeDtypeStruct((N_NODES, D), jnp.float32),
            jax.ShapeDtypeStruct((N_NODES, 1), jnp.float32),
        ],
    )(x, w_conv, degt)


def _tc2_body(accp_ref, y_ref, dinv_ref, bc_ref, wf_ref, bf_ref,
              wf2_ref, bf2_ref, out_ref):
    a = accp_ref[0] + accp_ref[1] + y_ref[...]
    h = jnp.maximum(a * dinv_ref[...] + bc_ref[...], 0.0)
    h = jnp.maximum(
        jnp.dot(h, wf_ref[...], preferred_element_type=jnp.float32)
        + bf_ref[...], 0.0)
    out_ref[...] = (
        jnp.dot(h, wf2_ref[...], preferred_element_type=jnp.float32)
        + bf2_ref[...])


def _tc2(accp, y, dinv, b_conv, w_fc, b_fc, w_fc2, b_fc2):
    R = 2000
    grid = N_NODES // R
    wspec = pl.BlockSpec((D, D), lambda i: (0, 0))
    bspec = pl.BlockSpec((1, D), lambda i: (0, 0))
    return pl.pallas_call(
        _tc2_body,
        grid=(grid,),
        in_specs=[
            pl.BlockSpec((NC, R, D), lambda i: (0, i, 0)),
            pl.BlockSpec((R, D), lambda i: (i, 0)),
            pl.BlockSpec((R, 1), lambda i: (i, 0)),
            bspec, wspec, bspec, wspec, bspec,
        ],
        out_specs=pl.BlockSpec((R, D), lambda i: (i, 0)),
        out_shape=jax.ShapeDtypeStruct((N_NODES, D), jnp.float32),
    )(accp, y, dinv, b_conv, w_fc, b_fc, w_fc2, b_fc2)


def kernel(g, x, edge_weight, W_conv, b_conv, W_fc, b_fc, W_fc2, b_fc2):
    row2d = g[0].astype(jnp.int32).reshape(N_CHUNKS, K)
    col2d = g[1].astype(jnp.int32).reshape(N_CHUNKS, K)
    ew2d = edge_weight.astype(jnp.float32).reshape(N_CHUNKS, K)

    zeros_n = jnp.zeros((N_NODES,), jnp.float32)
    zeros_nd = jnp.zeros((N_NODES, D), jnp.float32)

    degp = _sc_degree(col2d, ew2d, zeros_n)
    degt = degp.T
    y, dinv = _tc1(x, W_conv, degt)
    accp = _sc_edge(row2d, col2d, ew2d, y, zeros_nd)
    out = _tc2(accp, y, dinv,
               b_conv.reshape(1, D), W_fc, b_fc.reshape(1, D),
               W_fc2, b_fc2.reshape(1, D))
    return out


# speedup vs baseline: 31.2399x; 2.1244x over previous
import functools

import jax
import jax.numpy as jnp
from jax import lax
from jax.experimental import pallas as pl
from jax.experimental.pallas import tpu as pltpu
from jax.experimental.pallas import tpu_sc as plsc

N_NODES = 10000
N_EDGES = 320000
D = 128

NC = 2
NS = 16
NW = NC * NS

K = 128
N_CHUNKS = N_EDGES // K
CH = N_CHUNKS // NW
N_EXTRA = N_CHUNKS - CH * NW
RPS = N_NODES // NS
PH = 26
NPH = CH // PH


def _sc_mesh():
    return plsc.VectorSubcoreMesh(core_axis_name="c", subcore_axis_name="s")


_SC_PARAMS = pltpu.CompilerParams(use_tc_tiling_on_sc=False,
                                  needs_layout_passes=False)


def _sc_deg_body(col3, ew3, colx, ewx, zeros_n, degp, colb, ewb, sem, deg_sh):
    c = lax.axis_index("c")
    s = lax.axis_index("s")
    wid = c * NS + s

    pltpu.sync_copy(col3.at[wid], colb.at[pl.ds(0, CH)])
    pltpu.sync_copy(ew3.at[wid], ewb.at[pl.ds(0, CH)])

    @pl.when(wid < N_EXTRA)
    def _extra_load():
        pltpu.sync_copy(colx.at[wid], colb.at[CH])
        pltpu.sync_copy(ewx.at[wid], ewb.at[CH])

    @pl.when(s == 0)
    def _init():
        pltpu.sync_copy(zeros_n, deg_sh)

    plsc.subcore_barrier()

    WIN = 8

    def fire(j, carry):
        pltpu.async_copy(ewb.at[j], deg_sh.at[colb.at[j]], sem, add=True)

        @pl.when(j >= WIN)
        def _():
            pltpu.make_async_copy(ewb.at[j], deg_sh.at[colb.at[j]],
                                  sem).wait()

        return carry

    lax.fori_loop(0, CH, fire, 0)
    for _ in range(min(WIN, CH)):
        pltpu.make_async_copy(ewb.at[0], deg_sh.at[colb.at[0]], sem).wait()

    @pl.when(wid < N_EXTRA)
    def _extra_fire():
        pltpu.async_copy(ewb.at[CH], deg_sh.at[colb.at[CH]], sem, add=True)
        pltpu.make_async_copy(ewb.at[CH], deg_sh.at[colb.at[CH]], sem).wait()

    plsc.subcore_barrier()

    @pl.when(s == 0)
    def _writeout():
        pltpu.sync_copy(deg_sh, degp.at[c])


def _sc_degree(col3, ew3, colx, ewx, zeros_n):
    return pl.kernel(
        _sc_deg_body,
        out_type=jax.ShapeDtypeStruct((NC, N_NODES), jnp.float32),
        mesh=_sc_mesh(),
        scratch_types=[
            pltpu.VMEM((CH + 1, K), jnp.int32),
            pltpu.VMEM((CH + 1, K), jnp.float32),
            pltpu.SemaphoreType.DMA,
            pltpu.VMEM_SHARED((N_NODES,), jnp.float32),
        ],
        compiler_params=_SC_PARAMS,
    )(col3, ew3, colx, ewx, zeros_n)


def _scale_chunk(buf_ref, ew_ref):
    def group(u, carry):
        for t in range(8):
            e = u * 8 + t
            w = plsc.load_gather(ew_ref, [jnp.full((16,), e, jnp.int32)])
            for d in range(D // 16):
                buf_ref[e, pl.ds(d * 16, 16)] = (
                    buf_ref[e, pl.ds(d * 16, 16)] * w)
        return carry

    lax.fori_loop(0, K // 8, group, 0)


def _sc_edge_body(row3, col3, ew3, rowx, colx, ewx, y_hbm, zeros_nd, accp,
                  rowb, colb, ewb, buf0, buf1, gsem, ssem, acc_sh):
    c = lax.axis_index("c")
    s = lax.axis_index("s")
    wid = c * NS + s

    pltpu.sync_copy(zeros_nd.at[pl.ds(s * RPS, RPS)],
                    acc_sh.at[pl.ds(s * RPS, RPS)])
    plsc.subcore_barrier()

    def step(j, buf_a, buf_b):
        @pl.when(j > 0)
        def _():
            pltpu.make_async_copy(buf_b, acc_sh.at[colb.at[0]], ssem).wait()

        @pl.when(j + 1 < PH)
        def _():
            pltpu.async_copy(y_hbm.at[rowb.at[j + 1]], buf_b, gsem)

        pltpu.make_async_copy(y_hbm.at[rowb.at[0]], buf_a, gsem).wait()
        _scale_chunk(buf_a, ewb.at[j])
        pltpu.async_copy(buf_a, acc_sh.at[colb.at[j]], ssem, add=True)

    def pair(t, carry):
        step(2 * t, buf0, buf1)
        step(2 * t + 1, buf1, buf0)
        return carry

    for p in range(NPH):
        pltpu.sync_copy(row3.at[wid, pl.ds(p * PH, PH)], rowb.at[pl.ds(0, PH)])
        pltpu.sync_copy(col3.at[wid, pl.ds(p * PH, PH)], colb.at[pl.ds(0, PH)])
        pltpu.sync_copy(ew3.at[wid, pl.ds(p * PH, PH)], ewb.at[pl.ds(0, PH)])
        pltpu.async_copy(y_hbm.at[rowb.at[0]], buf0, gsem)
        lax.fori_loop(0, PH // 2, pair, 0)
        pltpu.make_async_copy(buf1, acc_sh.at[colb.at[0]], ssem).wait()

    @pl.when(wid < N_EXTRA)
    def _extra_chunk():
        pltpu.sync_copy(rowx.at[wid], rowb.at[0])
        pltpu.sync_copy(colx.at[wid], colb.at[0])
        pltpu.sync_copy(ewx.at[wid], ewb.at[0])
        pltpu.async_copy(y_hbm.at[rowb.at[0]], buf0, gsem)
        pltpu.make_async_copy(y_hbm.at[rowb.at[0]], buf0, gsem).wait()
        _scale_chunk(buf0, ewb.at[0])
        pltpu.async_copy(buf0, acc_sh.at[colb.at[0]], ssem, add=True)
        pltpu.make_async_copy(buf0, acc_sh.at[colb.at[0]], ssem).wait()

    plsc.subcore_barrier()

    pltpu.sync_copy(acc_sh.at[pl.ds(s * RPS, RPS)],
                    accp.at[c, pl.ds(s * RPS, RPS)])


def _sc_edge(row3, col3, ew3, rowx, colx, ewx, y, zeros_nd):
    return pl.kernel(
        _sc_edge_body,
        out_type=jax.ShapeDtypeStruct((NC, N_NODES, D), jnp.float32),
        mesh=_sc_mesh(),
        scratch_types=[
            pltpu.VMEM((PH, K), jnp.int32),
            pltpu.VMEM((PH, K), jnp.int32),
            pltpu.VMEM((PH, K), jnp.float32),
            pltpu.VMEM((K, D), jnp.float32),
            pltpu.VMEM((K, D), jnp.float32),
            pltpu.SemaphoreType.DMA,
            pltpu.SemaphoreType.DMA,
            pltpu.VMEM_SHARED((N_NODES, D), jnp.float32),
        ],
        compiler_params=_SC_PARAMS,
    )(row3, col3, ew3, rowx, colx, ewx, y, zeros_nd)


def _tc1_body(x_ref, w_ref, degt_ref, y_ref, dinv_ref):
    deg = degt_ref[:, 0:1] + degt_ref[:, 1:2] + 1.0
    dinv = lax.rsqrt(deg)
    xw = jnp.dot(x_ref[...], w_ref[...], preferred_element_type=jnp.float32)
    y_ref[...] = xw * dinv
    dinv_ref[...] = dinv


def _tc1(x, w_conv, degt):
    R = 2000
    grid = N_NODES // R
    return pl.pallas_call(
        _tc1_body,
        grid=(grid,),
        in_specs=[
            pl.BlockSpec((R, D), lambda i: (i, 0)),
            pl.BlockSpec((D, D), lambda i: (0, 0)),
            pl.BlockSpec((R, NC), lambda i: (i, 0)),
        ],
        out_specs=[
            pl.BlockSpec((R, D), lambda i: (i, 0)),
            pl.BlockSpec((R, 1), lambda i: (i, 0)),
        ],
        out_shape=[
            jax.ShapeDtypeStruct((N_NODES, D), jnp.float32),
            jax.ShapeDtypeStruct((N_NODES, 1), jnp.float32),
        ],
    )(x, w_conv, degt)


def _tc2_body(accp_ref, y_ref, dinv_ref, bc_ref, wf_ref, bf_ref,
              wf2_ref, bf2_ref, out_ref):
    a = accp_ref[0] + accp_ref[1] + y_ref[...]
    h = jnp.maximum(a * dinv_ref[...] + bc_ref[...], 0.0)
    h = jnp.maximum(
        jnp.dot(h, wf_ref[...], preferred_element_type=jnp.float32)
        + bf_ref[...], 0.0)
    out_ref[...] = (
        jnp.dot(h, wf2_ref[...], preferred_element_type=jnp.float32)
        + bf2_ref[...])


def _tc2(accp, y, dinv, b_conv, w_fc, b_fc, w_fc2, b_fc2):
    R = 2000
    grid = N_NODES // R
    wspec = pl.BlockSpec((D, D), lambda i: (0, 0))
    bspec = pl.BlockSpec((1, D), lambda i: (0, 0))
    return pl.pallas_call(
        _tc2_body,
        grid=(grid,),
        in_specs=[
            pl.BlockSpec((NC, R, D), lambda i: (0, i, 0)),
            pl.BlockSpec((R, D), lambda i: (i, 0)),
            pl.BlockSpec((R, 1), lambda i: (i, 0)),
            bspec, wspec, bspec, wspec, bspec,
        ],
        out_specs=pl.BlockSpec((R, D), lambda i: (i, 0)),
        out_shape=jax.ShapeDtypeStruct((N_NODES, D), jnp.float32),
    )(accp, y, dinv, b_conv, w_fc, b_fc, w_fc2, b_fc2)


def kernel(g, x, edge_weight, W_conv, b_conv, W_fc, b_fc, W_fc2, b_fc2):
    row2d = g[0].astype(jnp.int32).reshape(N_CHUNKS, K)
    col2d = g[1].astype(jnp.int32).reshape(N_CHUNKS, K)
    ew2d = edge_weight.astype(jnp.float32).reshape(N_CHUNKS, K)
    nmain = CH * NW
    row3, rowx = row2d[:nmain].reshape(NW, CH, K), row2d[nmain:]
    col3, colx = col2d[:nmain].reshape(NW, CH, K), col2d[nmain:]
    ew3, ewx = ew2d[:nmain].reshape(NW, CH, K), ew2d[nmain:]

    zeros_n = jnp.zeros((N_NODES,), jnp.float32)
    zeros_nd = jnp.zeros((N_NODES, D), jnp.float32)

    degp = _sc_degree(col3, ew3, colx, ewx, zeros_n)
    degt = degp.T
    y, dinv = _tc1(x, W_conv, degt)
    accp = _sc_edge(row3, col3, ew3, rowx, colx, ewx, y, zeros_nd)
    out = _tc2(accp, y, dinv,
               b_conv.reshape(1, D), W_fc, b_fc.reshape(1, D),
               W_fc2, b_fc2.reshape(1, D))
    return out


# speedup vs baseline: 34.3384x; 1.0992x over previous
import functools

import jax
import jax.numpy as jnp
from jax import lax
from jax.experimental import pallas as pl
from jax.experimental.pallas import tpu as pltpu
from jax.experimental.pallas import tpu_sc as plsc

N_NODES = 10000
N_EDGES = 320000
D = 128

NC = 2
NS = 16
NW = NC * NS

K = 128
N_CHUNKS = N_EDGES // K
CH = N_CHUNKS // NW
N_EXTRA = N_CHUNKS - CH * NW
RPS = N_NODES // NS
PH = 26
NPH = CH // PH


def _sc_mesh():
    return plsc.VectorSubcoreMesh(core_axis_name="c", subcore_axis_name="s")


_SC_PARAMS = pltpu.CompilerParams(use_tc_tiling_on_sc=False,
                                  needs_layout_passes=False)


def _sc_deg_body(col3, ew3, colx, ewx, zeros_n, degp, colb, ewb, sem, deg_sh):
    c = lax.axis_index("c")
    s = lax.axis_index("s")
    wid = c * NS + s

    pltpu.sync_copy(col3.at[wid], colb.at[pl.ds(0, CH)])
    pltpu.sync_copy(ew3.at[wid], ewb.at[pl.ds(0, CH)])

    @pl.when(wid < N_EXTRA)
    def _extra_load():
        pltpu.sync_copy(colx.at[wid], colb.at[CH])
        pltpu.sync_copy(ewx.at[wid], ewb.at[CH])

    @pl.when(s == 0)
    def _init():
        pltpu.sync_copy(zeros_n, deg_sh)

    plsc.subcore_barrier()

    WIN = 8

    def fire(j, carry):
        pltpu.async_copy(ewb.at[j], deg_sh.at[colb.at[j]], sem, add=True)

        @pl.when(j >= WIN)
        def _():
            pltpu.make_async_copy(ewb.at[j], deg_sh.at[colb.at[j]],
                                  sem).wait()

        return carry

    lax.fori_loop(0, CH, fire, 0)
    for _ in range(min(WIN, CH)):
        pltpu.make_async_copy(ewb.at[0], deg_sh.at[colb.at[0]], sem).wait()

    @pl.when(wid < N_EXTRA)
    def _extra_fire():
        pltpu.async_copy(ewb.at[CH], deg_sh.at[colb.at[CH]], sem, add=True)
        pltpu.make_async_copy(ewb.at[CH], deg_sh.at[colb.at[CH]], sem).wait()

    plsc.subcore_barrier()

    @pl.when(s == 0)
    def _writeout():
        pltpu.sync_copy(deg_sh, degp.at[c])


def _sc_degree(col3, ew3, colx, ewx, zeros_n):
    return pl.kernel(
        _sc_deg_body,
        out_type=jax.ShapeDtypeStruct((NC, N_NODES), jnp.float32),
        mesh=_sc_mesh(),
        scratch_types=[
            pltpu.VMEM((CH + 1, K), jnp.int32),
            pltpu.VMEM((CH + 1, K), jnp.float32),
            pltpu.SemaphoreType.DMA,
            pltpu.VMEM_SHARED((N_NODES,), jnp.float32),
        ],
        compiler_params=_SC_PARAMS,
    )(col3, ew3, colx, ewx, zeros_n)


def _scale_chunk(buf_ref, ew_ref):
    @plsc.parallel_loop(0, K, 1, unroll=8)
    def _(e):
        w = plsc.load_gather(ew_ref, [jnp.full((16,), e, jnp.int32)])
        for d in range(D // 16):
            buf_ref[e, pl.ds(d * 16, 16)] = (
                buf_ref[e, pl.ds(d * 16, 16)] * w)


def _sc_edge_body(row3, col3, ew3, rowx, colx, ewx, y_hbm, zeros_nd, accp,
                  rowb, colb, ewb, buf0, buf1, gsem, ssem, acc_sh):
    c = lax.axis_index("c")
    s = lax.axis_index("s")
    wid = c * NS + s

    pltpu.sync_copy(zeros_nd.at[pl.ds(s * RPS, RPS)],
                    acc_sh.at[pl.ds(s * RPS, RPS)])
    plsc.subcore_barrier()

    def step(j, buf_a, buf_b):
        @pl.when(j > 0)
        def _():
            pltpu.make_async_copy(buf_b, acc_sh.at[colb.at[0]], ssem).wait()

        @pl.when(j + 1 < PH)
        def _():
            pltpu.async_copy(y_hbm.at[rowb.at[j + 1]], buf_b, gsem)

        pltpu.make_async_copy(y_hbm.at[rowb.at[0]], buf_a, gsem).wait()
        _scale_chunk(buf_a, ewb.at[j])
        pltpu.async_copy(buf_a, acc_sh.at[colb.at[j]], ssem, add=True)

    def pair(t, carry):
        step(2 * t, buf0, buf1)
        step(2 * t + 1, buf1, buf0)
        return carry

    for p in range(NPH):
        pltpu.sync_copy(row3.at[wid, pl.ds(p * PH, PH)], rowb.at[pl.ds(0, PH)])
        pltpu.sync_copy(col3.at[wid, pl.ds(p * PH, PH)], colb.at[pl.ds(0, PH)])
        pltpu.sync_copy(ew3.at[wid, pl.ds(p * PH, PH)], ewb.at[pl.ds(0, PH)])
        pltpu.async_copy(y_hbm.at[rowb.at[0]], buf0, gsem)
        lax.fori_loop(0, PH // 2, pair, 0)
        pltpu.make_async_copy(buf1, acc_sh.at[colb.at[0]], ssem).wait()

    @pl.when(wid < N_EXTRA)
    def _extra_chunk():
        pltpu.sync_copy(rowx.at[wid], rowb.at[0])
        pltpu.sync_copy(colx.at[wid], colb.at[0])
        pltpu.sync_copy(ewx.at[wid], ewb.at[0])
        pltpu.async_copy(y_hbm.at[rowb.at[0]], buf0, gsem)
        pltpu.make_async_copy(y_hbm.at[rowb.at[0]], buf0, gsem).wait()
        _scale_chunk(buf0, ewb.at[0])
        pltpu.async_copy(buf0, acc_sh.at[colb.at[0]], ssem, add=True)
        pltpu.make_async_copy(buf0, acc_sh.at[colb.at[0]], ssem).wait()

    plsc.subcore_barrier()

    pltpu.sync_copy(acc_sh.at[pl.ds(s * RPS, RPS)],
                    accp.at[c, pl.ds(s * RPS, RPS)])


def _sc_edge(row3, col3, ew3, rowx, colx, ewx, y, zeros_nd):
    return pl.kernel(
        _sc_edge_body,
        out_type=jax.ShapeDtypeStruct((NC, N_NODES, D), jnp.float32),
        mesh=_sc_mesh(),
        scratch_types=[
            pltpu.VMEM((PH, K), jnp.int32),
            pltpu.VMEM((PH, K), jnp.int32),
            pltpu.VMEM((PH, K), jnp.float32),
            pltpu.VMEM((K, D), jnp.float32),
            pltpu.VMEM((K, D), jnp.float32),
            pltpu.SemaphoreType.DMA,
            pltpu.SemaphoreType.DMA,
            pltpu.VMEM_SHARED((N_NODES, D), jnp.float32),
        ],
        compiler_params=_SC_PARAMS,
    )(row3, col3, ew3, rowx, colx, ewx, y, zeros_nd)


def _tc1_body(x_ref, w_ref, degp_ref, y_ref, dinv_ref):
    deg = degp_ref[0] + degp_ref[1] + 1.0
    dinv = lax.rsqrt(deg)
    xw = jnp.dot(x_ref[...], w_ref[...], preferred_element_type=jnp.float32)
    y_ref[...] = xw * dinv
    dinv_ref[...] = dinv


def _tc1(x, w_conv, degp3):
    R = 2000
    grid = N_NODES // R
    return pl.pallas_call(
        _tc1_body,
        grid=(grid,),
        in_specs=[
            pl.BlockSpec((R, D), lambda i: (i, 0)),
            pl.BlockSpec((D, D), lambda i: (0, 0)),
            pl.BlockSpec((NC, R, 1), lambda i: (0, i, 0)),
        ],
        out_specs=[
            pl.BlockSpec((R, D), lambda i: (i, 0)),
            pl.BlockSpec((R, 1), lambda i: (i, 0)),
        ],
        out_shape=[
            jax.ShapeDtypeStruct((N_NODES, D), jnp.float32),
            jax.ShapeDtypeStruct((N_NODES, 1), jnp.float32),
        ],
    )(x, w_conv, degp3)


def _tc2_body(accp_ref, y_ref, dinv_ref, bc_ref, wf_ref, bf_ref,
              wf2_ref, bf2_ref, out_ref):
    a = accp_ref[0] + accp_ref[1] + y_ref[...]
    h = jnp.maximum(a * dinv_ref[...] + bc_ref[...], 0.0)
    h = jnp.maximum(
        jnp.dot(h, wf_ref[...], preferred_element_type=jnp.float32)
        + bf_ref[...], 0.0)
    out_ref[...] = (
        jnp.dot(h, wf2_ref[...], preferred_element_type=jnp.float32)
        + bf2_ref[...])


def _tc2(accp, y, dinv, b_conv, w_fc, b_fc, w_fc2, b_fc2):
    R = 2000
    grid = N_NODES // R
    wspec = pl.BlockSpec((D, D), lambda i: (0, 0))
    bspec = pl.BlockSpec((1, D), lambda i: (0, 0))
    return pl.pallas_call(
        _tc2_body,
        grid=(grid,),
        in_specs=[
            pl.BlockSpec((NC, R, D), lambda i: (0, i, 0)),
            pl.BlockSpec((R, D), lambda i: (i, 0)),
            pl.BlockSpec((R, 1), lambda i: (i, 0)),
            bspec, wspec, bspec, wspec, bspec,
        ],
        out_specs=pl.BlockSpec((R, D), lambda i: (i, 0)),
        out_shape=jax.ShapeDtypeStruct((N_NODES, D), jnp.float32),
    )(accp, y, dinv, b_conv, w_fc, b_fc, w_fc2, b_fc2)


def kernel(g, x, edge_weight, W_conv, b_conv, W_fc, b_fc, W_fc2, b_fc2):
    row2d = g[0].astype(jnp.int32).reshape(N_CHUNKS, K)
    col2d = g[1].astype(jnp.int32).reshape(N_CHUNKS, K)
    ew2d = edge_weight.astype(jnp.float32).reshape(N_CHUNKS, K)
    nmain = CH * NW
    row3, rowx = row2d[:nmain].reshape(NW, CH, K), row2d[nmain:]
    col3, colx = col2d[:nmain].reshape(NW, CH, K), col2d[nmain:]
    ew3, ewx = ew2d[:nmain].reshape(NW, CH, K), ew2d[nmain:]

    zeros_n = jnp.zeros((N_NODES,), jnp.float32)
    zeros_nd = jnp.zeros((N_NODES, D), jnp.float32)

    degp = _sc_degree(col3, ew3, colx, ewx, zeros_n)
    y, dinv = _tc1(x, W_conv, degp.reshape(NC, N_NODES, 1))
    accp = _sc_edge(row3, col3, ew3, rowx, colx, ewx, y, zeros_nd)
    out = _tc2(accp, y, dinv,
               b_conv.reshape(1, D), W_fc, b_fc.reshape(1, D),
               W_fc2, b_fc2.reshape(1, D))
    return out
